# Initial kernel scaffold; baseline (speedup 1.0000x reference)
import jax
import jax.numpy as jnp
from jax.experimental import pallas as pl


def kernel(x, edge_index, batch, W1, b1, W2, b2, Wlin, blin):
    raise NotImplementedError("write your pallas kernel here")


# speedup vs baseline: 11.6010x; 11.6010x over previous
import jax
import jax.numpy as jnp
from jax import lax
from jax.experimental import pallas as pl
from jax.experimental.pallas import tpu as pltpu
from jax.experimental.pallas import tpu_sc as plsc

N = 10000
E = 160000
D = 256
H = 256
C = 40
G = 128

NC = 2
NS = 16
HALF = D // 2
NP = 10240

EC = 80
CHUNKS_C = E // NS // EC
AC = 40
CHUNKS_A = E // (NC * NS) // AC
STRIPE = NP // NS

_PREC = jax.lax.Precision.HIGHEST


def _mesh():
    return plsc.VectorSubcoreMesh(
        core_axis_name="c", subcore_axis_name="s", num_cores=NC, num_subcores=NS
    )


def _deg_body(dst_hbm, out_hbm, ones_vm, zbuf, dstbuf, sem, acc):
    c = lax.axis_index("c")
    s = lax.axis_index("s")
    w = s * NC + c

    for i in range(AC):
        ones_vm[i, :] = jnp.ones((16,), jnp.float32)
    for i in range(128):
        zbuf[i, :] = jnp.zeros((16,), jnp.float32)
    for k in range(STRIPE // 128):
        pltpu.sync_copy(zbuf, acc.at[pl.ds(s * STRIPE + k * 128, 128)])
    plsc.subcore_barrier()

    pltpu.sync_copy(dst_hbm.at[w], dstbuf)

    def step(j, carry):
        pltpu.sync_copy(ones_vm, acc.at[dstbuf.at[j]], add=True)
        return carry

    lax.fori_loop(0, CHUNKS_A, step, 0)
    plsc.subcore_barrier()
    pltpu.sync_copy(acc.at[pl.ds(s * STRIPE, STRIPE)],
                    out_hbm.at[c, pl.ds(s * STRIPE, STRIPE)])


def _deg_counts(dst_r):
    return pl.kernel(
        _deg_body,
        out_type=jax.ShapeDtypeStruct((NC, NP, 16), jnp.float32),
        mesh=_mesh(),
        scratch_types=[
            pltpu.VMEM((AC, 16), jnp.float32),
            pltpu.VMEM((128, 16), jnp.float32),
            pltpu.VMEM((CHUNKS_A, AC), jnp.int32),
            pltpu.SemaphoreType.DMA,
            pltpu.VMEM_SHARED((NP, 16), jnp.float32),
        ],
        name="sc_deg",
    )(dst_r)


def _agg_body(lo_hbm, hi_hbm, src_hbm, dst_hbm, olo_hbm, ohi_hbm,
              srcbuf, dstbuf, rows, sem, acc):
    c = lax.axis_index("c")
    s = lax.axis_index("s")
    stripe = pl.ds(s * STRIPE, STRIPE)

    pltpu.sync_copy(src_hbm.at[s], srcbuf)
    pltpu.sync_copy(dst_hbm.at[s], dstbuf)

    def run(h_hbm, o_hbm):
        pltpu.sync_copy(h_hbm.at[stripe], acc.at[stripe])
        plsc.subcore_barrier()

        def step(j, carry):
            pltpu.async_copy(h_hbm.at[srcbuf.at[j]], rows, sem).wait()
            pltpu.sync_copy(rows, acc.at[dstbuf.at[j]], add=True)
            return carry

        lax.fori_loop(0, CHUNKS_C, step, 0)
        plsc.subcore_barrier()
        pltpu.sync_copy(acc.at[stripe], o_hbm.at[stripe])

    @pl.when(c == 0)
    def _():
        run(lo_hbm, olo_hbm)

    @pl.when(c == 1)
    def _():
        run(hi_hbm, ohi_hbm)


def _edge_aggregate(slo, shi, src_r, dst_r):
    return pl.kernel(
        _agg_body,
        out_type=(
            jax.ShapeDtypeStruct((NP, HALF), jnp.float32),
            jax.ShapeDtypeStruct((NP, HALF), jnp.float32),
        ),
        mesh=_mesh(),
        scratch_types=[
            pltpu.VMEM((CHUNKS_C, EC), jnp.int32),
            pltpu.VMEM((CHUNKS_C, EC), jnp.int32),
            pltpu.VMEM((EC, HALF), jnp.float32),
            pltpu.SemaphoreType.DMA,
            pltpu.VMEM_SHARED((NP, HALF), jnp.float32),
        ],
        name="sc_edge_agg",
    )(slo, shi, src_r, dst_r)


BN = 1024
NSTEPS_P = NP // BN
BN3 = 1000
NSTEPS_3 = N // BN3


def _dinv(d0_ref, d1_ref):
    deg = d0_ref[:, 0:1] + d1_ref[:, 0:1] + 1.0
    return lax.rsqrt(deg)


def _b1_body(x_ref, w_ref, d0_ref, d1_ref, slo_ref, shi_ref):
    dinv = _dinv(d0_ref, d1_ref)
    xw = jnp.dot(x_ref[...], w_ref[...], precision=_PREC,
                 preferred_element_type=jnp.float32)
    scaled = xw * dinv
    slo_ref[...] = scaled[:, :HALF]
    shi_ref[...] = scaled[:, HALF:]


def _b1(x, W1, deg0, deg1):
    return pl.pallas_call(
        _b1_body,
        grid=(NSTEPS_P,),
        in_specs=[
            pl.BlockSpec((BN, D), lambda i: (i, 0)),
            pl.BlockSpec((D, H), lambda i: (0, 0)),
            pl.BlockSpec((BN, 16), lambda i: (i, 0)),
            pl.BlockSpec((BN, 16), lambda i: (i, 0)),
        ],
        out_specs=[
            pl.BlockSpec((BN, HALF), lambda i: (i, 0)),
            pl.BlockSpec((BN, HALF), lambda i: (i, 0)),
        ],
        out_shape=[
            jax.ShapeDtypeStruct((NP, HALF), jnp.float32),
            jax.ShapeDtypeStruct((NP, HALF), jnp.float32),
        ],
        name="tc_b1",
    )(x, W1, deg0, deg1)


def _b2_body(alo_ref, ahi_ref, w_ref, b_ref, d0_ref, d1_ref,
             slo_ref, shi_ref):
    dinv = _dinv(d0_ref, d1_ref)
    agg = jnp.concatenate([alo_ref[...], ahi_ref[...]], axis=1)
    h = jnp.maximum(agg * dinv + b_ref[...], 0.0)
    xw = jnp.dot(h, w_ref[...], precision=_PREC,
                 preferred_element_type=jnp.float32)
    scaled = xw * dinv
    slo_ref[...] = scaled[:, :HALF]
    shi_ref[...] = scaled[:, HALF:]


def _b2(alo, ahi, W2, b1, deg0, deg1):
    return pl.pallas_call(
        _b2_body,
        grid=(NSTEPS_P,),
        in_specs=[
            pl.BlockSpec((BN, HALF), lambda i: (i, 0)),
            pl.BlockSpec((BN, HALF), lambda i: (i, 0)),
            pl.BlockSpec((H, H), lambda i: (0, 0)),
            pl.BlockSpec((1, H), lambda i: (0, 0)),
            pl.BlockSpec((BN, 16), lambda i: (i, 0)),
            pl.BlockSpec((BN, 16), lambda i: (i, 0)),
        ],
        out_specs=[
            pl.BlockSpec((BN, HALF), lambda i: (i, 0)),
            pl.BlockSpec((BN, HALF), lambda i: (i, 0)),
        ],
        out_shape=[
            jax.ShapeDtypeStruct((NP, HALF), jnp.float32),
            jax.ShapeDtypeStruct((NP, HALF), jnp.float32),
        ],
        name="tc_b2",
    )(alo, ahi, W2, b1, deg0, deg1)


def _b3_body(alo_ref, ahi_ref, b_ref, d0_ref, d1_ref, batch_ref,
             wlin_ref, blin_ref, out_ref, e_ref):
    k = pl.program_id(0)
    dinv = _dinv(d0_ref, d1_ref)
    agg = jnp.concatenate([alo_ref[...], ahi_ref[...]], axis=1)
    h = agg * dinv + b_ref[...]
    gid = lax.broadcasted_iota(jnp.int32, (BN3, G), 1)
    onehot = (batch_ref[...] == gid).astype(jnp.float32)
    part = lax.dot_general(onehot, h, (((0,), (0,)), ((), ())),
                           precision=_PREC,
                           preferred_element_type=jnp.float32)

    @pl.when(k == 0)
    def _():
        e_ref[...] = jnp.zeros_like(e_ref)

    e_ref[...] += part

    @pl.when(k == NSTEPS_3 - 1)
    def _():
        out_ref[...] = jnp.dot(e_ref[...], wlin_ref[...], precision=_PREC,
                               preferred_element_type=jnp.float32) + blin_ref[...]


def _b3(alo, ahi, b2, deg0, deg1, batch2d, Wlin, blin):
    return pl.pallas_call(
        _b3_body,
        grid=(NSTEPS_3,),
        in_specs=[
            pl.BlockSpec((BN3, HALF), lambda i: (i, 0)),
            pl.BlockSpec((BN3, HALF), lambda i: (i, 0)),
            pl.BlockSpec((1, H), lambda i: (0, 0)),
            pl.BlockSpec((BN3, 16), lambda i: (i, 0)),
            pl.BlockSpec((BN3, 16), lambda i: (i, 0)),
            pl.BlockSpec((BN3, 1), lambda i: (i, 0)),
            pl.BlockSpec((H, C), lambda i: (0, 0)),
            pl.BlockSpec((1, C), lambda i: (0, 0)),
        ],
        out_specs=[
            pl.BlockSpec((G, C), lambda i: (0, 0)),
            pl.BlockSpec((G, H), lambda i: (0, 0)),
        ],
        out_shape=[
            jax.ShapeDtypeStruct((G, C), jnp.float32),
            jax.ShapeDtypeStruct((G, H), jnp.float32),
        ],
        name="tc_b3",
    )(alo, ahi, b2, deg0, deg1, batch2d, Wlin, blin)


@jax.jit
def kernel(x, edge_index, batch, W1, b1, W2, b2, Wlin, blin):
    src = edge_index[0]
    dst = edge_index[1]
    src_r = src.reshape(NS, CHUNKS_C, EC)
    dst_r = dst.reshape(NS, CHUNKS_C, EC)
    dst_deg = dst.reshape(NC * NS, CHUNKS_A, AC)
    batch2d = batch.reshape(N, 1)
    b1r = b1.reshape(1, H)
    b2r = b2.reshape(1, H)
    blinr = blin.reshape(1, C)

    degp = _deg_counts(dst_deg)
    deg0 = degp[0]
    deg1 = degp[1]

    s1lo, s1hi = _b1(x, W1, deg0, deg1)
    a1lo, a1hi = _edge_aggregate(s1lo, s1hi, src_r, dst_r)
    s2lo, s2hi = _b2(a1lo, a1hi, W2, b1r, deg0, deg1)
    a2lo, a2hi = _edge_aggregate(s2lo, s2hi, src_r, dst_r)
    out, e = _b3(a2lo, a2hi, b2r, deg0, deg1, batch2d, Wlin, blinr)
    return (out, e)
